# baseline (speedup 1.0000x reference)
import functools

import jax
import jax.numpy as jnp
from jax import lax
from jax.experimental import pallas as pl
from jax.experimental.pallas import tpu as pltpu
from jax.experimental.pallas import tpu_sc as plsc

N, M, B = 4096, 8192, 16
C_IN, C_SKIP = 256, 128
K = 3
H1, H2 = 512, 256
C_CAT = C_IN + C_SKIP

NW = 32
CH = M // NW
GP = 32
NG = CH // GP

MB = 512
GRID1 = M // MB

_INF = float("inf")
_BIGI = 2147483647


def _sc_interp_body(px_h, py_h, pz_h, bc_h, qx_h, qy_h, qz_h, bq_h, lo_h,
                    hi_h, x_h, y_h,
                    pxv, pyv, pzv, bcv, qxv, qyv, qzv, bqv, lov, hiv,
                    idxv, wv, rows0, rows1, yv, sem, gsem0, gsem1):
    cid = lax.axis_index("c")
    sid = lax.axis_index("s")
    wid = sid * 2 + cid
    base = wid * CH

    pltpu.sync_copy(px_h, pxv)
    pltpu.sync_copy(py_h, pyv)
    pltpu.sync_copy(pz_h, pzv)
    pltpu.sync_copy(bc_h, bcv)
    pltpu.sync_copy(qx_h.at[pl.ds(base, CH)], qxv)
    pltpu.sync_copy(qy_h.at[pl.ds(base, CH)], qyv)
    pltpu.sync_copy(qz_h.at[pl.ds(base, CH)], qzv)
    pltpu.sync_copy(bq_h.at[pl.ds(base, CH)], bqv)
    pltpu.sync_copy(lo_h, lov)
    pltpu.sync_copy(hi_h, hiv)

    iota = lax.broadcasted_iota(jnp.int32, (16,), 0)
    mask3 = iota < 3

    def merge_weights_store(p, sel, qxs, qys, qzs):
        b1, b2, b3, i1, i2, i3 = sel
        cands = []
        for _ in range(K):
            m = jnp.min(b1)
            cnd = jnp.min(jnp.where(b1 == m, i1, _BIGI))
            hit = (b1 == m) & (i1 == cnd)
            b1 = jnp.where(hit, b2, b1)
            i1 = jnp.where(hit, i2, i1)
            b2 = jnp.where(hit, b3, b2)
            i2 = jnp.where(hit, i3, i2)
            b3 = jnp.where(hit, _INF, b3)
            cands.append(cnd)

        cvec = jnp.where(iota == 0, cands[0],
                         jnp.where(iota == 1, cands[1], cands[2]))
        gx = plsc.load_gather(pxv, [cvec])
        gy = plsc.load_gather(pyv, [cvec])
        gz = plsc.load_gather(pzv, [cvec])
        dx = gx - qxs
        dy = gy - qys
        dz = gz - qzs
        d2f = (dx * dx + dy * dy) + dz * dz
        w = 1.0 / jnp.maximum(d2f, 1e-16)
        wm = jnp.where(mask3, w, 0.0)
        vn = wm / jnp.sum(wm)
        slot = p * 3 + iota
        plsc.store_scatter(idxv, [slot], cvec, mask=mask3)
        plsc.store_scatter(wv, [slot], vn, mask=mask3)

    def _insert(car, d, jv):
        b1, b2, b3, i1, i2, i3 = car
        c1 = d < b1
        c2 = d < b2
        c3 = d < b3
        t1 = jnp.maximum(b1, d)
        t2 = jnp.maximum(b2, t1)
        nb1 = jnp.minimum(b1, d)
        nb2 = jnp.minimum(b2, t1)
        nb3 = jnp.minimum(b3, t2)
        ni1 = jnp.where(c1, jv, i1)
        ni2 = jnp.where(c1, i1, jnp.where(c2, jv, i2))
        ni3 = jnp.where(c2, i2, jnp.where(c3, jv, i3))
        return nb1, nb2, nb3, ni1, ni2, ni3

    inf_v = jnp.full((16,), _INF, jnp.float32)
    big_v = jnp.full((16,), _BIGI, jnp.int32)
    init6 = (inf_v, inf_v, inf_v, big_v, big_v, big_v)

    def single_point(p):
        pv = jnp.full((16,), p, jnp.int32)
        qxs = plsc.load_gather(qxv, [pv])
        qys = plsc.load_gather(qyv, [pv])
        qzs = plsc.load_gather(qzv, [pv])
        bqs = plsc.load_gather(bqv, [pv])
        st = jnp.min(plsc.load_gather(lov, [bqs]))
        en = jnp.min(plsc.load_gather(hiv, [bqs]))
        fb = (en - st) < 3

        def scan(st2, en2, with_pen):
            nsteps = (en2 - st2 + 15) // 16

            def load_chunk(t):
                jv = (st2 + t * 16) + iota
                msk = jv < en2
                jvc = jnp.where(msk, jv, 0)
                cx = plsc.load_gather(pxv, [jvc])
                cy = plsc.load_gather(pyv, [jvc])
                cz = plsc.load_gather(pzv, [jvc])
                cb = plsc.load_gather(bcv, [jvc]) if with_pen else jvc
                return jv, msk, cx, cy, cz, cb

            def insert(car, chunk):
                jv, msk, cx, cy, cz, cb = chunk
                dx = cx - qxs
                dy = cy - qys
                dz = cz - qzs
                d2 = (dx * dx + dy * dy) + dz * dz
                if with_pen:
                    d2 = d2 + jnp.where(cb == bqs, 0.0, 1e10)
                d = jnp.where(msk, d2, _INF)
                return _insert(car, d, jv)

            def step(t, car):
                sel, chunk = car[:6], car[6:]
                nxt = load_chunk(t + 1)
                return insert(sel, chunk) + nxt

            init = init6 + load_chunk(0)
            car = lax.fori_loop(0, nsteps - 1, step, init)
            return insert(car[:6], car[6:])

        sel = lax.cond(
            fb,
            lambda: scan(0, N, True),
            lambda: scan(st, en, False))
        merge_weights_store(p, sel, qxs, qys, qzs)

    def pair_body(q, _):
        p0 = q * 2
        p1 = p0 + 1
        pv0 = jnp.full((16,), p0, jnp.int32)
        pv1 = jnp.full((16,), p1, jnp.int32)
        qxs0 = plsc.load_gather(qxv, [pv0])
        qys0 = plsc.load_gather(qyv, [pv0])
        qzs0 = plsc.load_gather(qzv, [pv0])
        qxs1 = plsc.load_gather(qxv, [pv1])
        qys1 = plsc.load_gather(qyv, [pv1])
        qzs1 = plsc.load_gather(qzv, [pv1])
        bqs0 = plsc.load_gather(bqv, [pv0])
        bqs1 = plsc.load_gather(bqv, [pv1])
        sts0 = plsc.load_gather(lov, [bqs0])
        ens0 = plsc.load_gather(hiv, [bqs0])
        sts1 = plsc.load_gather(lov, [bqs1])
        ens1 = plsc.load_gather(hiv, [bqs1])
        lo = jnp.min(jnp.minimum(sts0, sts1))
        hi = jnp.max(jnp.maximum(ens0, ens1))
        fbv = ((ens0 - sts0) < 3) | ((ens1 - sts1) < 3)
        fb = jnp.max(fbv.astype(jnp.int32)) == 1

        def slow():
            single_point(p0)
            single_point(p1)
            return 0

        def fast():
            nsteps = (hi - lo + 15) // 16

            def load_chunk(t):
                jv = (lo + t * 16) + iota
                jvc = jnp.where(jv < hi, jv, 0)
                cx = plsc.load_gather(pxv, [jvc])
                cy = plsc.load_gather(pyv, [jvc])
                cz = plsc.load_gather(pzv, [jvc])
                return jv, cx, cy, cz

            def insert2(car, chunk):
                jv, cx, cy, cz = chunk
                dx0 = cx - qxs0
                dy0 = cy - qys0
                dz0 = cz - qzs0
                d20 = (dx0 * dx0 + dy0 * dy0) + dz0 * dz0
                d0 = jnp.where((jv >= sts0) & (jv < ens0), d20, _INF)
                dx1 = cx - qxs1
                dy1 = cy - qys1
                dz1 = cz - qzs1
                d21 = (dx1 * dx1 + dy1 * dy1) + dz1 * dz1
                d1 = jnp.where((jv >= sts1) & (jv < ens1), d21, _INF)
                return (_insert(car[:6], d0, jv) + _insert(car[6:], d1, jv))

            def step(t, car):
                sel, chunk = car[:12], car[12:]
                nxt = load_chunk(t + 1)
                return insert2(sel, chunk) + nxt

            init = init6 + init6 + load_chunk(0)
            car = lax.fori_loop(0, nsteps - 1, step, init)
            final = insert2(car[:12], car[12:])
            merge_weights_store(p0, final[:6], qxs0, qys0, qzs0)
            merge_weights_store(p1, final[6:], qxs1, qys1, qzs1)
            return 0

        lax.cond(fb, slow, fast)
        return 0

    def combine_group(g, cur):
        def pbody(p2, _):
            p = p2 * 2
            wloc = jnp.full((16,), (g * GP + p) * 3, jnp.int32)
            wa0 = plsc.load_gather(wv, [wloc])
            wa1 = plsc.load_gather(wv, [wloc + 1])
            wa2 = plsc.load_gather(wv, [wloc + 2])
            wb0 = plsc.load_gather(wv, [wloc + 3])
            wb1 = plsc.load_gather(wv, [wloc + 4])
            wb2 = plsc.load_gather(wv, [wloc + 5])
            r0 = p * 3
            for cg in range(C_IN // 16):
                sl = pl.ds(cg * 16, 16)
                yv[p, sl] = ((cur[r0, sl] * wa0 + cur[r0 + 1, sl] * wa1)
                             + cur[r0 + 2, sl] * wa2)
                yv[p + 1, sl] = ((cur[r0 + 3, sl] * wb0
                                  + cur[r0 + 4, sl] * wb1)
                                 + cur[r0 + 5, sl] * wb2)
            return 0

        lax.fori_loop(0, GP // 2, pbody, 0)
        pltpu.sync_copy(yv, y_h.at[pl.ds(base + g * GP, GP), :])

    rows = [rows0, rows1]
    sems = [gsem0, gsem1]
    for g in range(NG):
        with jax.named_scope("nn_search"):
            lax.fori_loop(g * (GP // 2), (g + 1) * (GP // 2), pair_body, 0)
        pltpu.async_copy(x_h.at[idxv.at[pl.ds(g * 3 * GP, 3 * GP)]],
                         rows[g % 2], sems[g % 2])
        if g > 0:
            with jax.named_scope("dwait"):
                pltpu.make_async_copy(
                    x_h.at[idxv.at[pl.ds((g - 1) * 3 * GP, 3 * GP)]],
                    rows[(g - 1) % 2], sems[(g - 1) % 2]).wait()
            with jax.named_scope("combine"):
                combine_group(g - 1, rows[(g - 1) % 2])
    with jax.named_scope("dwait"):
        pltpu.make_async_copy(
            x_h.at[idxv.at[pl.ds((NG - 1) * 3 * GP, 3 * GP)]],
            rows[(NG - 1) % 2], sems[(NG - 1) % 2]).wait()
    with jax.named_scope("combine"):
        combine_group(NG - 1, rows[(NG - 1) % 2])


def _sc_interpolate(pos, batch_c, pos_skip, batch_f, seg_lo, seg_hi, x):
    mesh = plsc.VectorSubcoreMesh(core_axis_name="c", subcore_axis_name="s")
    f32, i32 = jnp.float32, jnp.int32
    run = pl.kernel(
        _sc_interp_body,
        mesh=mesh,
        compiler_params=pltpu.CompilerParams(needs_layout_passes=False),
        out_type=jax.ShapeDtypeStruct((M, C_IN), f32),
        scratch_types=[
            pltpu.VMEM((N,), f32), pltpu.VMEM((N,), f32),
            pltpu.VMEM((N,), f32), pltpu.VMEM((N,), i32),
            pltpu.VMEM((CH,), f32), pltpu.VMEM((CH,), f32),
            pltpu.VMEM((CH,), f32),
            pltpu.VMEM((CH,), i32),
            pltpu.VMEM((B,), i32), pltpu.VMEM((B,), i32),
            pltpu.VMEM((3 * CH,), i32),
            pltpu.VMEM((3 * CH,), f32),
            pltpu.VMEM((3 * GP, C_IN), f32),
            pltpu.VMEM((3 * GP, C_IN), f32),
            pltpu.VMEM((GP, C_IN), f32),
            pltpu.SemaphoreType.DMA,
            pltpu.SemaphoreType.DMA,
            pltpu.SemaphoreType.DMA,
        ],
    )
    return run(pos[:, 0], pos[:, 1], pos[:, 2], batch_c,
               pos_skip[:, 0], pos_skip[:, 1], pos_skip[:, 2], batch_f,
               seg_lo, seg_hi, x)


def _leaky(h):
    return jnp.where(h >= 0, h, 0.01 * h)


def _bn_affine(s, q, g, be):
    mu = s / M
    var = q / M - mu * mu
    a = g * lax.rsqrt(var + 1e-5)
    c = be - mu * a
    return a, c


def _mlp_body(y_ref, xs_ref, w1a_ref, w1b_ref, b1_ref, w2_ref, b2_ref,
              g1_ref, be1_ref, g2_ref, be2_ref, h_ref,
              z1s, z2s, s1s, q1s, s2s, q2s):
    i = pl.program_id(0)
    r = lax.rem(i, GRID1)
    rows = pl.ds(r * MB, MB)

    @pl.when(i == 0)
    def _():
        s1s[:] = jnp.zeros_like(s1s)
        q1s[:] = jnp.zeros_like(q1s)
        s2s[:] = jnp.zeros_like(s2s)
        q2s[:] = jnp.zeros_like(q2s)

    @pl.when(i < GRID1)
    def _():
        z1 = (jnp.dot(y_ref[:], w1a_ref[:],
                      preferred_element_type=jnp.float32)
              + jnp.dot(xs_ref[:], w1b_ref[:],
                        preferred_element_type=jnp.float32)
              + b1_ref[:])
        z1 = _leaky(z1)
        z1s[rows, :] = z1
        s1s[:] += jnp.sum(z1, axis=0, keepdims=True)
        q1s[:] += jnp.sum(z1 * z1, axis=0, keepdims=True)

    @pl.when((i >= GRID1) & (i < 2 * GRID1))
    def _():
        a1, c1 = _bn_affine(s1s[:], q1s[:], g1_ref[:], be1_ref[:])
        z2 = (jnp.dot(z1s[rows, :] * a1 + c1, w2_ref[:],
                      preferred_element_type=jnp.float32) + b2_ref[:])
        z2 = _leaky(z2)
        z2s[rows, :] = z2
        s2s[:] += jnp.sum(z2, axis=0, keepdims=True)
        q2s[:] += jnp.sum(z2 * z2, axis=0, keepdims=True)

    @pl.when(i >= 2 * GRID1)
    def _():
        a2, c2 = _bn_affine(s2s[:], q2s[:], g2_ref[:], be2_ref[:])
        h_ref[:] = z2s[rows, :] * a2 + c2


@jax.jit
def kernel(x, pos, batch, x_skip, pos_skip, batch_skip,
           W1, b1, g1, be1, W2, b2, g2, be2):
    bc = batch.astype(jnp.int32)
    bf = batch_skip.astype(jnp.int32)
    bins = jnp.arange(B, dtype=jnp.int32)
    seg_lo = jnp.sum(bc[None, :] < bins[:, None], axis=1).astype(jnp.int32)
    seg_hi = jnp.sum(bc[None, :] <= bins[:, None], axis=1).astype(jnp.int32)

    y = _sc_interpolate(pos, bc, pos_skip, bf, seg_lo, seg_hi, x)

    w1a = W1[:, :C_IN].T
    w1b = W1[:, C_IN:].T
    w2t = W2.T

    def _fine_map(i):
        return (jnp.minimum(i, GRID1 - 1), 0)

    h = pl.pallas_call(
        _mlp_body,
        grid=(3 * GRID1,),
        in_specs=[
            pl.BlockSpec((MB, C_IN), _fine_map),
            pl.BlockSpec((MB, C_SKIP), _fine_map),
            pl.BlockSpec((C_IN, H1), lambda i: (0, 0)),
            pl.BlockSpec((C_SKIP, H1), lambda i: (0, 0)),
            pl.BlockSpec((1, H1), lambda i: (0, 0)),
            pl.BlockSpec((H1, H2), lambda i: (0, 0)),
            pl.BlockSpec((1, H2), lambda i: (0, 0)),
            pl.BlockSpec((1, H1), lambda i: (0, 0)),
            pl.BlockSpec((1, H1), lambda i: (0, 0)),
            pl.BlockSpec((1, H2), lambda i: (0, 0)),
            pl.BlockSpec((1, H2), lambda i: (0, 0)),
        ],
        out_specs=pl.BlockSpec(
            (MB, H2), lambda i: (jnp.maximum(i - 2 * GRID1, 0), 0)),
        out_shape=jax.ShapeDtypeStruct((M, H2), jnp.float32),
        scratch_shapes=[
            pltpu.VMEM((M, H1), jnp.float32),
            pltpu.VMEM((M, H2), jnp.float32),
            pltpu.VMEM((1, H1), jnp.float32),
            pltpu.VMEM((1, H1), jnp.float32),
            pltpu.VMEM((1, H2), jnp.float32),
            pltpu.VMEM((1, H2), jnp.float32),
        ],
        compiler_params=pltpu.CompilerParams(
            dimension_semantics=("arbitrary",)),
    )(y, x_skip, w1a, w1b, b1.reshape(1, H1), w2t, b2.reshape(1, H2),
      g1.reshape(1, H1), be1.reshape(1, H1),
      g2.reshape(1, H2), be2.reshape(1, H2))

    return (h, pos_skip, batch_skip)


# speedup vs baseline: 1.1323x; 1.1323x over previous
import functools

import jax
import jax.numpy as jnp
from jax import lax
from jax.experimental import pallas as pl
from jax.experimental.pallas import tpu as pltpu
from jax.experimental.pallas import tpu_sc as plsc

N, M, B = 4096, 8192, 16
C_IN, C_SKIP = 256, 128
K = 3
H1, H2 = 512, 256
C_CAT = C_IN + C_SKIP

NW = 32
CH = M // NW
GP = 32
NG = CH // GP

MB = 512
GRID1 = M // MB

_INF = float("inf")
_BIGI = 2147483647


def _sc_interp_body(px_h, py_h, pz_h, bc_h, qx_h, qy_h, qz_h, bq_h, lo_h,
                    hi_h, x_h, y_h, w_h,
                    pxv, pyv, pzv, bcv, qxv, qyv, qzv, bqv, lov, hiv,
                    idxv, wv, rows0, rows1, gsem0, gsem1, osem0, osem1):
    cid = lax.axis_index("c")
    sid = lax.axis_index("s")
    wid = sid * 2 + cid
    base = wid * CH

    pltpu.sync_copy(px_h, pxv)
    pltpu.sync_copy(py_h, pyv)
    pltpu.sync_copy(pz_h, pzv)
    pltpu.sync_copy(bc_h, bcv)
    pltpu.sync_copy(qx_h.at[pl.ds(base, CH)], qxv)
    pltpu.sync_copy(qy_h.at[pl.ds(base, CH)], qyv)
    pltpu.sync_copy(qz_h.at[pl.ds(base, CH)], qzv)
    pltpu.sync_copy(bq_h.at[pl.ds(base, CH)], bqv)
    pltpu.sync_copy(lo_h, lov)
    pltpu.sync_copy(hi_h, hiv)

    iota = lax.broadcasted_iota(jnp.int32, (16,), 0)
    mask3 = iota < 3

    def merge_weights_store(p, sel, qxs, qys, qzs):
        b1, b2, b3, i1, i2, i3 = sel
        cands = []
        for _ in range(K):
            m = jnp.min(b1)
            cnd = jnp.min(jnp.where(b1 == m, i1, _BIGI))
            hit = (b1 == m) & (i1 == cnd)
            b1 = jnp.where(hit, b2, b1)
            i1 = jnp.where(hit, i2, i1)
            b2 = jnp.where(hit, b3, b2)
            i2 = jnp.where(hit, i3, i2)
            b3 = jnp.where(hit, _INF, b3)
            cands.append(cnd)

        cvec = jnp.where(iota == 0, cands[0],
                         jnp.where(iota == 1, cands[1], cands[2]))
        gx = plsc.load_gather(pxv, [cvec])
        gy = plsc.load_gather(pyv, [cvec])
        gz = plsc.load_gather(pzv, [cvec])
        dx = gx - qxs
        dy = gy - qys
        dz = gz - qzs
        d2f = (dx * dx + dy * dy) + dz * dz
        w = 1.0 / jnp.maximum(d2f, 1e-16)
        wm = jnp.where(mask3, w, 0.0)
        vn = wm / jnp.sum(wm)
        slot = p + iota * CH
        plsc.store_scatter(idxv, [slot], cvec, mask=mask3)
        plsc.store_scatter(wv, [slot], vn, mask=mask3)

    def _insert(car, d, jv):
        b1, b2, b3, i1, i2, i3 = car
        c1 = d < b1
        c2 = d < b2
        c3 = d < b3
        t1 = jnp.maximum(b1, d)
        t2 = jnp.maximum(b2, t1)
        nb1 = jnp.minimum(b1, d)
        nb2 = jnp.minimum(b2, t1)
        nb3 = jnp.minimum(b3, t2)
        ni1 = jnp.where(c1, jv, i1)
        ni2 = jnp.where(c1, i1, jnp.where(c2, jv, i2))
        ni3 = jnp.where(c2, i2, jnp.where(c3, jv, i3))
        return nb1, nb2, nb3, ni1, ni2, ni3

    inf_v = jnp.full((16,), _INF, jnp.float32)
    big_v = jnp.full((16,), _BIGI, jnp.int32)
    init6 = (inf_v, inf_v, inf_v, big_v, big_v, big_v)

    def single_point(p):
        pv = jnp.full((16,), p, jnp.int32)
        qxs = plsc.load_gather(qxv, [pv])
        qys = plsc.load_gather(qyv, [pv])
        qzs = plsc.load_gather(qzv, [pv])
        bqs = plsc.load_gather(bqv, [pv])
        st = jnp.min(plsc.load_gather(lov, [bqs]))
        en = jnp.min(plsc.load_gather(hiv, [bqs]))
        fb = (en - st) < 3

        def scan(st2, en2, with_pen):
            nsteps = (en2 - st2 + 15) // 16

            def load_chunk(t):
                jv = (st2 + t * 16) + iota
                msk = jv < en2
                jvc = jnp.where(msk, jv, 0)
                cx = plsc.load_gather(pxv, [jvc])
                cy = plsc.load_gather(pyv, [jvc])
                cz = plsc.load_gather(pzv, [jvc])
                cb = plsc.load_gather(bcv, [jvc]) if with_pen else jvc
                return jv, msk, cx, cy, cz, cb

            def insert(car, chunk):
                jv, msk, cx, cy, cz, cb = chunk
                dx = cx - qxs
                dy = cy - qys
                dz = cz - qzs
                d2 = (dx * dx + dy * dy) + dz * dz
                if with_pen:
                    d2 = d2 + jnp.where(cb == bqs, 0.0, 1e10)
                d = jnp.where(msk, d2, _INF)
                return _insert(car, d, jv)

            def step(t, car):
                sel, chunk = car[:6], car[6:]
                nxt = load_chunk(t + 1)
                return insert(sel, chunk) + nxt

            init = init6 + load_chunk(0)
            car = lax.fori_loop(0, nsteps - 1, step, init)
            return insert(car[:6], car[6:])

        sel = lax.cond(
            fb,
            lambda: scan(0, N, True),
            lambda: scan(st, en, False))
        merge_weights_store(p, sel, qxs, qys, qzs)

    def pair_body(q, _):
        p0 = q * 2
        p1 = p0 + 1
        pv0 = jnp.full((16,), p0, jnp.int32)
        pv1 = jnp.full((16,), p1, jnp.int32)
        qxs0 = plsc.load_gather(qxv, [pv0])
        qys0 = plsc.load_gather(qyv, [pv0])
        qzs0 = plsc.load_gather(qzv, [pv0])
        qxs1 = plsc.load_gather(qxv, [pv1])
        qys1 = plsc.load_gather(qyv, [pv1])
        qzs1 = plsc.load_gather(qzv, [pv1])
        bqs0 = plsc.load_gather(bqv, [pv0])
        bqs1 = plsc.load_gather(bqv, [pv1])
        sts0 = plsc.load_gather(lov, [bqs0])
        ens0 = plsc.load_gather(hiv, [bqs0])
        sts1 = plsc.load_gather(lov, [bqs1])
        ens1 = plsc.load_gather(hiv, [bqs1])
        lo = jnp.min(jnp.minimum(sts0, sts1))
        hi = jnp.max(jnp.maximum(ens0, ens1))
        fbv = ((ens0 - sts0) < 3) | ((ens1 - sts1) < 3)
        fb = jnp.max(fbv.astype(jnp.int32)) == 1

        def slow():
            single_point(p0)
            single_point(p1)
            return 0

        def fast():
            nsteps = (hi - lo + 15) // 16

            def load_chunk(t):
                jv = (lo + t * 16) + iota
                jvc = jnp.where(jv < hi, jv, 0)
                cx = plsc.load_gather(pxv, [jvc])
                cy = plsc.load_gather(pyv, [jvc])
                cz = plsc.load_gather(pzv, [jvc])
                return jv, cx, cy, cz

            def insert2(car, chunk):
                jv, cx, cy, cz = chunk
                dx0 = cx - qxs0
                dy0 = cy - qys0
                dz0 = cz - qzs0
                d20 = (dx0 * dx0 + dy0 * dy0) + dz0 * dz0
                d0 = jnp.where((jv >= sts0) & (jv < ens0), d20, _INF)
                dx1 = cx - qxs1
                dy1 = cy - qys1
                dz1 = cz - qzs1
                d21 = (dx1 * dx1 + dy1 * dy1) + dz1 * dz1
                d1 = jnp.where((jv >= sts1) & (jv < ens1), d21, _INF)
                return (_insert(car[:6], d0, jv) + _insert(car[6:], d1, jv))

            def step(t, car):
                sel, chunk = car[:12], car[12:]
                nxt = load_chunk(t + 1)
                return insert2(sel, chunk) + nxt

            init = init6 + init6 + load_chunk(0)
            car = lax.fori_loop(0, nsteps - 1, step, init)
            final = insert2(car[:12], car[12:])
            merge_weights_store(p0, final[:6], qxs0, qys0, qzs0)
            merge_weights_store(p1, final[6:], qxs1, qys1, qzs1)
            return 0

        lax.cond(fb, slow, fast)
        return 0

    def gathers(g, buf, sem):
        cps = []
        for k in range(K):
            cps.append(pltpu.make_async_copy(
                x_h.at[idxv.at[pl.ds(k * CH + g * GP, GP)]],
                buf.at[pl.ds(k * GP, GP), :], sem))
        return cps

    def out_copies(g, buf, sem):
        cps = []
        for k in range(K):
            cps.append(pltpu.make_async_copy(
                buf.at[pl.ds(k * GP, GP), :],
                y_h.at[pl.ds(k * M + base + g * GP, GP), :], sem))
        return cps

    rows = [rows0, rows1]
    gsems = [gsem0, gsem1]
    osems = [osem0, osem1]
    for g in range(NG):
        with jax.named_scope("nn_search"):
            lax.fori_loop(g * (GP // 2), (g + 1) * (GP // 2), pair_body, 0)
        if g >= 2:
            for cp in out_copies(g - 2, rows[g % 2], osems[g % 2]):
                cp.wait()
        for cp in gathers(g, rows[g % 2], gsems[g % 2]):
            cp.start()
        if g >= 1:
            for cp in gathers(g - 1, rows[(g - 1) % 2], gsems[(g - 1) % 2]):
                cp.wait()
            for cp in out_copies(g - 1, rows[(g - 1) % 2],
                                 osems[(g - 1) % 2]):
                cp.start()
    for cp in gathers(NG - 1, rows[(NG - 1) % 2], gsems[(NG - 1) % 2]):
        cp.wait()
    for cp in out_copies(NG - 1, rows[(NG - 1) % 2], osems[(NG - 1) % 2]):
        cp.start()
    for cp in out_copies(NG - 2, rows[(NG - 2) % 2], osems[(NG - 2) % 2]):
        cp.wait()
    for cp in out_copies(NG - 1, rows[(NG - 1) % 2], osems[(NG - 1) % 2]):
        cp.wait()
    pltpu.sync_copy(wv.at[pl.ds(0, CH)], w_h.at[pl.ds(base, CH)])
    pltpu.sync_copy(wv.at[pl.ds(CH, CH)], w_h.at[pl.ds(M + base, CH)])
    pltpu.sync_copy(wv.at[pl.ds(2 * CH, CH)],
                    w_h.at[pl.ds(2 * M + base, CH)])


def _sc_interpolate(pos, batch_c, pos_skip, batch_f, seg_lo, seg_hi, x):
    mesh = plsc.VectorSubcoreMesh(core_axis_name="c", subcore_axis_name="s")
    f32, i32 = jnp.float32, jnp.int32
    run = pl.kernel(
        _sc_interp_body,
        mesh=mesh,
        compiler_params=pltpu.CompilerParams(needs_layout_passes=False),
        out_type=[
            jax.ShapeDtypeStruct((3 * M, C_IN), f32),
            jax.ShapeDtypeStruct((3 * M,), f32),
        ],
        scratch_types=[
            pltpu.VMEM((N,), f32), pltpu.VMEM((N,), f32),
            pltpu.VMEM((N,), f32), pltpu.VMEM((N,), i32),
            pltpu.VMEM((CH,), f32), pltpu.VMEM((CH,), f32),
            pltpu.VMEM((CH,), f32),
            pltpu.VMEM((CH,), i32),
            pltpu.VMEM((B,), i32), pltpu.VMEM((B,), i32),
            pltpu.VMEM((3 * CH,), i32),
            pltpu.VMEM((3 * CH,), f32),
            pltpu.VMEM((3 * GP, C_IN), f32),
            pltpu.VMEM((3 * GP, C_IN), f32),
            pltpu.SemaphoreType.DMA,
            pltpu.SemaphoreType.DMA,
            pltpu.SemaphoreType.DMA,
            pltpu.SemaphoreType.DMA,
        ],
    )
    return run(pos[:, 0], pos[:, 1], pos[:, 2], batch_c,
               pos_skip[:, 0], pos_skip[:, 1], pos_skip[:, 2], batch_f,
               seg_lo, seg_hi, x)


def _leaky(h):
    return jnp.where(h >= 0, h, 0.01 * h)


def _bn_affine(s, q, g, be):
    mu = s / M
    var = q / M - mu * mu
    a = g * lax.rsqrt(var + 1e-5)
    c = be - mu * a
    return a, c


def _mlp_body(ya_ref, yb_ref, yc_ref, wa_ref, wb_ref, wc_ref, xs_ref,
              w1a_ref, w1b_ref, b1_ref, w2_ref,
              b2_ref, g1_ref, be1_ref, g2_ref, be2_ref, h_ref,
              z1s, z2s, s1s, q1s, s2s, q2s):
    i = pl.program_id(0)
    r = lax.rem(i, GRID1)
    rows = pl.ds(r * MB, MB)

    @pl.when(i == 0)
    def _():
        s1s[:] = jnp.zeros_like(s1s)
        q1s[:] = jnp.zeros_like(q1s)
        s2s[:] = jnp.zeros_like(s2s)
        q2s[:] = jnp.zeros_like(q2s)

    @pl.when(i < GRID1)
    def _():
        y = (ya_ref[:] * wa_ref[:] + yb_ref[:] * wb_ref[:]
             + yc_ref[:] * wc_ref[:])
        z1 = (jnp.dot(y, w1a_ref[:],
                      preferred_element_type=jnp.float32)
              + jnp.dot(xs_ref[:], w1b_ref[:],
                        preferred_element_type=jnp.float32)
              + b1_ref[:])
        z1 = _leaky(z1)
        z1s[rows, :] = z1
        s1s[:] += jnp.sum(z1, axis=0, keepdims=True)
        q1s[:] += jnp.sum(z1 * z1, axis=0, keepdims=True)

    @pl.when((i >= GRID1) & (i < 2 * GRID1))
    def _():
        a1, c1 = _bn_affine(s1s[:], q1s[:], g1_ref[:], be1_ref[:])
        z2 = (jnp.dot(z1s[rows, :] * a1 + c1, w2_ref[:],
                      preferred_element_type=jnp.float32) + b2_ref[:])
        z2 = _leaky(z2)
        z2s[rows, :] = z2
        s2s[:] += jnp.sum(z2, axis=0, keepdims=True)
        q2s[:] += jnp.sum(z2 * z2, axis=0, keepdims=True)

    @pl.when(i >= 2 * GRID1)
    def _():
        a2, c2 = _bn_affine(s2s[:], q2s[:], g2_ref[:], be2_ref[:])
        h_ref[:] = z2s[rows, :] * a2 + c2


@jax.jit
def kernel(x, pos, batch, x_skip, pos_skip, batch_skip,
           W1, b1, g1, be1, W2, b2, g2, be2):
    bc = batch.astype(jnp.int32)
    bf = batch_skip.astype(jnp.int32)
    bins = jnp.arange(B, dtype=jnp.int32)
    seg_lo = jnp.sum(bc[None, :] < bins[:, None], axis=1).astype(jnp.int32)
    seg_hi = jnp.sum(bc[None, :] <= bins[:, None], axis=1).astype(jnp.int32)

    y3, w3 = _sc_interpolate(pos, bc, pos_skip, bf, seg_lo, seg_hi, x)
    w3 = w3.reshape(3 * M, 1)

    w1a = W1[:, :C_IN].T
    w1b = W1[:, C_IN:].T
    w2t = W2.T

    def _fine_map(i):
        return (jnp.minimum(i, GRID1 - 1), 0)

    def _plane_map(k):
        return lambda i: (k * GRID1 + jnp.minimum(i, GRID1 - 1), 0)

    h = pl.pallas_call(
        _mlp_body,
        grid=(3 * GRID1,),
        in_specs=[
            pl.BlockSpec((MB, C_IN), _plane_map(0)),
            pl.BlockSpec((MB, C_IN), _plane_map(1)),
            pl.BlockSpec((MB, C_IN), _plane_map(2)),
            pl.BlockSpec((MB, 1), _plane_map(0)),
            pl.BlockSpec((MB, 1), _plane_map(1)),
            pl.BlockSpec((MB, 1), _plane_map(2)),
            pl.BlockSpec((MB, C_SKIP), _fine_map),
            pl.BlockSpec((C_IN, H1), lambda i: (0, 0)),
            pl.BlockSpec((C_SKIP, H1), lambda i: (0, 0)),
            pl.BlockSpec((1, H1), lambda i: (0, 0)),
            pl.BlockSpec((H1, H2), lambda i: (0, 0)),
            pl.BlockSpec((1, H2), lambda i: (0, 0)),
            pl.BlockSpec((1, H1), lambda i: (0, 0)),
            pl.BlockSpec((1, H1), lambda i: (0, 0)),
            pl.BlockSpec((1, H2), lambda i: (0, 0)),
            pl.BlockSpec((1, H2), lambda i: (0, 0)),
        ],
        out_specs=pl.BlockSpec(
            (MB, H2), lambda i: (jnp.maximum(i - 2 * GRID1, 0), 0)),
        out_shape=jax.ShapeDtypeStruct((M, H2), jnp.float32),
        scratch_shapes=[
            pltpu.VMEM((M, H1), jnp.float32),
            pltpu.VMEM((M, H2), jnp.float32),
            pltpu.VMEM((1, H1), jnp.float32),
            pltpu.VMEM((1, H1), jnp.float32),
            pltpu.VMEM((1, H2), jnp.float32),
            pltpu.VMEM((1, H2), jnp.float32),
        ],
        compiler_params=pltpu.CompilerParams(
            dimension_semantics=("arbitrary",)),
    )(y3, y3, y3, w3, w3, w3, x_skip, w1a, w1b,
      b1.reshape(1, H1), w2t, b2.reshape(1, H2),
      g1.reshape(1, H1), be1.reshape(1, H1),
      g2.reshape(1, H2), be2.reshape(1, H2))

    return (h, pos_skip, batch_skip)
